# initial kernel scaffold (unmeasured)
import jax
import jax.numpy as jnp
from jax import lax
from jax.experimental import pallas as pl
from jax.experimental.pallas import tpu as pltpu

N_DEV = 4
SQ = 2048
SKV = 2048
DM = 1024
HC = 8
DH = 128
BQ = 256
GW = 128
WIN = 512
NQB = SQ // BQ
SCALE = 0.08838834764831843


def kernel(x, Wq, K_ext, V_ext, Wo):
    x2 = x.reshape(SQ, DM)

    def body(x_ref, wq_ref, k_hbm, v_hbm, wo_ref, out_ref,
             comm, kbuf, vbuf, ctxc, send_sems, recv_sems, kv_sems):
        my = lax.axis_index("i")
        left = lax.rem(my + N_DEV - 1, N_DEV)
        right = lax.rem(my + 1, N_DEV)

        barrier = pltpu.get_barrier_semaphore()
        for nbr in (left, right):
            pl.semaphore_signal(barrier, inc=1, device_id=(nbr,),
                                device_id_type=pl.DeviceIdType.MESH)
        pl.semaphore_wait(barrier, 2)

        comm[0, :DM, :] = wq_ref[...]
        comm[0, DM:, :] = wo_ref[...]

        def kv_copies(h, slot):
            j = lax.rem(my - h + N_DEV, N_DEV)
            ck = pltpu.make_async_copy(
                k_hbm.at[my, :, pl.ds(j * HC, HC), :], kbuf.at[slot],
                kv_sems.at[slot, 0])
            cv = pltpu.make_async_copy(
                v_hbm.at[my, :, pl.ds(j * HC, HC), :], vbuf.at[slot],
                kv_sems.at[slot, 1])
            return ck, cv

        ck, cv = kv_copies(0, 0)
        ck.start()
        cv.start()
        pending = (ck, cv)

        r0 = lax.broadcasted_iota(jnp.int32, (BQ, SKV), 0)
        c0 = lax.broadcasted_iota(jnp.int32, (BQ, SKV), 1)
        m0 = (jnp.abs(r0 - c0) <= 128) | (c0 < 32) | (r0 < 32)
        bias0 = jnp.where(m0, 0.0, -1e9).astype(jnp.float32)

        for h in range(N_DEV):
            slot = h % 2
            if h < N_DEV - 1:
                rdma = pltpu.make_async_remote_copy(
                    src_ref=comm.at[slot],
                    dst_ref=comm.at[1 - slot],
                    send_sem=send_sems.at[slot],
                    recv_sem=recv_sems.at[1 - slot],
                    device_id=(right,),
                    device_id_type=pl.DeviceIdType.MESH,
                )
                rdma.start()

            pending[0].wait()
            pending[1].wait()
            if h < N_DEV - 1:
                nk, nv = kv_copies(h + 1, 1 - slot)
                nk.start()
                nv.start()
                pending = (nk, nv)

            wq_c = comm[slot, :DM, :]
            wo_c = comm[slot, DM:, :]
            q = lax.dot_general(x_ref[...], wq_c, (((1,), (0,)), ((), ())),
                                preferred_element_type=jnp.float32) * SCALE

            for hd in range(HC):
                qh = q[:BQ, hd * DH:(hd + 1) * DH]
                kh = kbuf[slot, :, hd, :]
                s = lax.dot_general(qh, kh, (((1,), (1,)), ((), ())),
                                    preferred_element_type=jnp.float32)
                s = s + bias0
                mx = jnp.max(s, axis=1, keepdims=True)
                w = jnp.exp(s - mx)
                denom = jnp.sum(w, axis=1, keepdims=True)
                vh = vbuf[slot, :, hd, :]
                ctx = lax.dot_general(w, vh, (((1,), (0,)), ((), ())),
                                      preferred_element_type=jnp.float32)
                ctxc[:BQ, hd * DH:(hd + 1) * DH] = ctx / denom

            def qb_body(qb, carry):
                qs = qb * BQ
                start = jnp.minimum(qs - 128, SKV - WIN)
                rr = lax.broadcasted_iota(jnp.int32, (BQ, GW + WIN), 0) + qs
                cc = lax.broadcasted_iota(jnp.int32, (BQ, GW + WIN), 1)
                ki = jnp.where(cc < GW, cc, start + (cc - GW))
                m = jnp.where(cc < GW, cc < 32, jnp.abs(rr - ki) <= 128)
                bias = jnp.where(m, 0.0, -1e9).astype(jnp.float32)
                for hd in range(HC):
                    qh = lax.dynamic_slice(q, (qs, hd * DH), (BQ, DH))
                    khg = kbuf[slot, :GW, hd, :]
                    khw = kbuf[slot, pl.ds(start, WIN), hd, :]
                    kh = jnp.concatenate([khg, khw], axis=0)
                    s = lax.dot_general(qh, kh, (((1,), (1,)), ((), ())),
                                        preferred_element_type=jnp.float32)
                    s = s + bias
                    mx = jnp.max(s, axis=1, keepdims=True)
                    w = jnp.exp(s - mx)
                    denom = jnp.sum(w, axis=1, keepdims=True)
                    vhg = vbuf[slot, :GW, hd, :]
                    vhw = vbuf[slot, pl.ds(start, WIN), hd, :]
                    vh = jnp.concatenate([vhg, vhw], axis=0)
                    ctx = lax.dot_general(w, vh, (((1,), (0,)), ((), ())),
                                          preferred_element_type=jnp.float32)
                    ctxc[pl.ds(qs, BQ), hd * DH:(hd + 1) * DH] = ctx / denom
                return carry

            lax.fori_loop(1, NQB, qb_body, 0)

            contrib = lax.dot_general(ctxc[...], wo_c, (((1,), (0,)), ((), ())),
                                      preferred_element_type=jnp.float32)
            if h == 0:
                out_ref[...] = contrib
            else:
                out_ref[...] = out_ref[...] + contrib

            if h < N_DEV - 1:
                rdma.wait()

    out2 = pl.pallas_call(
        body,
        out_shape=jax.ShapeDtypeStruct((SQ, DM), jnp.float32),
        in_specs=[
            pl.BlockSpec(memory_space=pltpu.VMEM),
            pl.BlockSpec(memory_space=pltpu.VMEM),
            pl.BlockSpec(memory_space=pltpu.ANY),
            pl.BlockSpec(memory_space=pltpu.ANY),
            pl.BlockSpec(memory_space=pltpu.VMEM),
        ],
        out_specs=pl.BlockSpec(memory_space=pltpu.VMEM),
        scratch_shapes=[
            pltpu.VMEM((2, 2 * DM, DM), jnp.float32),
            pltpu.VMEM((2, SKV, HC, DH), jnp.float32),
            pltpu.VMEM((2, SKV, HC, DH), jnp.float32),
            pltpu.VMEM((SQ, DM), jnp.float32),
            pltpu.SemaphoreType.DMA((2,)),
            pltpu.SemaphoreType.DMA((2,)),
            pltpu.SemaphoreType.DMA((2, 2)),
        ],
        compiler_params=pltpu.CompilerParams(collective_id=0),
    )(x2, Wq, K_ext, V_ext, Wo)
    return out2.reshape(1, SQ, DM)


# baseline (device time: 356758 ns/iter reference)
import jax
import jax.numpy as jnp
from jax import lax
from jax.experimental import pallas as pl
from jax.experimental.pallas import tpu as pltpu

N_DEV = 4
SQ = 2048
SKV = 2048
DM = 1024
HC = 8
DH = 128
BQ = 256
GW = 128
WIN = 512
NQB = SQ // BQ
SCALE = 0.08838834764831843


def kernel(x, Wq, K_ext, V_ext, Wo):
    x2 = x.reshape(SQ, DM)

    def body(x_ref, wq_hbm, k_hbm, v_hbm, wo_hbm, out_ref,
             comm, kbuf, vbuf, kwin, vwin, ctxb, bias0b,
             send_sems, recv_sems, kv_sems, w_sems):
        my = lax.axis_index("i")
        left = lax.rem(my + N_DEV - 1, N_DEV)
        right = lax.rem(my + 1, N_DEV)

        cw = pltpu.make_async_copy(wq_hbm, comm.at[0, :DM, :], w_sems.at[0])
        co = pltpu.make_async_copy(wo_hbm, comm.at[0, DM:, :], w_sems.at[1])
        cw.start()
        co.start()

        def kv_copies(h):
            j = lax.rem(my - h + N_DEV, N_DEV)
            cps = []
            for hd in range(HC):
                cps.append(pltpu.make_async_copy(
                    k_hbm.at[my, :, j * HC + hd, :], kbuf.at[hd],
                    kv_sems.at[0]))
                cps.append(pltpu.make_async_copy(
                    v_hbm.at[my, :, j * HC + hd, :], vbuf.at[hd],
                    kv_sems.at[1]))
            return cps

        kv_pend = kv_copies(0)
        for cp in kv_pend:
            cp.start()

        barrier = pltpu.get_barrier_semaphore()
        for nbr in (left, right):
            pl.semaphore_signal(barrier, inc=1, device_id=(nbr,),
                                device_id_type=pl.DeviceIdType.MESH)
        pl.semaphore_wait(barrier, 2)

        cw.wait()
        co.wait()
        for cp in kv_pend:
            cp.wait()

        r0 = lax.broadcasted_iota(jnp.int32, (BQ, SKV), 0)
        c0 = lax.broadcasted_iota(jnp.int32, (BQ, SKV), 1)
        m0 = (jnp.abs(r0 - c0) <= 128) | (c0 < 32) | (r0 < 32)
        bias0b[...] = jnp.where(m0, 0.0, -1e9).astype(jnp.float32)

        def attend(qh, k_sl, v_sl, bias):
            s = lax.dot_general(qh, k_sl, (((1,), (1,)), ((), ())),
                                preferred_element_type=jnp.float32)
            s = s + bias
            mx = jnp.max(s, axis=1, keepdims=True)
            w = jnp.exp(s - mx)
            denom = jnp.sum(w, axis=1, keepdims=True)
            ctx = lax.dot_general(w, v_sl, (((1,), (0,)), ((), ())),
                                  preferred_element_type=jnp.float32)
            return ctx / denom

        for h in range(N_DEV):
            slot = h % 2
            if h < N_DEV - 1:
                rdma = pltpu.make_async_remote_copy(
                    src_ref=comm.at[slot],
                    dst_ref=comm.at[1 - slot],
                    send_sem=send_sems.at[slot],
                    recv_sem=recv_sems.at[1 - slot],
                    device_id=(right,),
                    device_id_type=pl.DeviceIdType.MESH,
                )
                rdma.start()

            def accum(qs):
                contrib = lax.dot_general(
                    ctxb[...], comm[slot, DM:, :], (((1,), (0,)), ((), ())),
                    preferred_element_type=jnp.float32)
                if h == 0:
                    out_ref[pl.ds(qs, BQ), :] = contrib
                else:
                    out_ref[pl.ds(qs, BQ), :] = (
                        out_ref[pl.ds(qs, BQ), :] + contrib)

            q0 = lax.dot_general(
                x_ref[:BQ, :], comm[slot, :DM, :], (((1,), (0,)), ((), ())),
                preferred_element_type=jnp.float32) * SCALE
            for hd in range(HC):
                ctxb[:, hd * DH:(hd + 1) * DH] = attend(
                    q0[:, hd * DH:(hd + 1) * DH], kbuf[hd], vbuf[hd],
                    bias0b[...])
            accum(0)

            def qb_body(qb, carry):
                qs = qb * BQ
                start = jnp.minimum(qs - 128, SKV - WIN)
                rr = lax.broadcasted_iota(jnp.int32, (BQ, GW + WIN), 0) + qs
                cc = lax.broadcasted_iota(jnp.int32, (BQ, GW + WIN), 1)
                ki = start + (cc - GW)
                m = (cc < 32) | ((cc >= GW) & (jnp.abs(rr - ki) <= 128))
                bias = jnp.where(m, 0.0, -1e9).astype(jnp.float32)
                q_blk = lax.dot_general(
                    x_ref[pl.ds(qs, BQ), :], comm[slot, :DM, :],
                    (((1,), (0,)), ((), ())),
                    preferred_element_type=jnp.float32) * SCALE
                for hd in range(HC):
                    kwin[:GW, :] = kbuf[hd, :GW, :]
                    kwin[GW:, :] = kbuf[hd, pl.ds(start, WIN), :]
                    vwin[:GW, :] = vbuf[hd, :GW, :]
                    vwin[GW:, :] = vbuf[hd, pl.ds(start, WIN), :]
                    ctxb[:, hd * DH:(hd + 1) * DH] = attend(
                        q_blk[:, hd * DH:(hd + 1) * DH],
                        kwin[...], vwin[...], bias)
                accum(qs)
                return carry

            lax.fori_loop(1, NQB, qb_body, 0)

            if h < N_DEV - 1:
                rdma.wait()
                kv_pend = kv_copies(h + 1)
                for cp in kv_pend:
                    cp.start()
                for cp in kv_pend:
                    cp.wait()

    out2 = pl.pallas_call(
        body,
        out_shape=jax.ShapeDtypeStruct((SQ, DM), jnp.float32),
        in_specs=[
            pl.BlockSpec(memory_space=pltpu.VMEM),
            pl.BlockSpec(memory_space=pl.ANY),
            pl.BlockSpec(memory_space=pl.ANY),
            pl.BlockSpec(memory_space=pl.ANY),
            pl.BlockSpec(memory_space=pl.ANY),
        ],
        out_specs=pl.BlockSpec(memory_space=pltpu.VMEM),
        scratch_shapes=[
            pltpu.VMEM((2, 2 * DM, DM), jnp.float32),
            pltpu.VMEM((HC, SKV, DH), jnp.float32),
            pltpu.VMEM((HC, SKV, DH), jnp.float32),
            pltpu.VMEM((GW + WIN, DH), jnp.float32),
            pltpu.VMEM((GW + WIN, DH), jnp.float32),
            pltpu.VMEM((BQ, DM), jnp.float32),
            pltpu.VMEM((BQ, SKV), jnp.float32),
            pltpu.SemaphoreType.DMA((2,)),
            pltpu.SemaphoreType.DMA((2,)),
            pltpu.SemaphoreType.DMA((2,)),
            pltpu.SemaphoreType.DMA((2,)),
        ],
        compiler_params=pltpu.CompilerParams(
            collective_id=0,
            vmem_limit_bytes=100 * 1024 * 1024,
        ),
    )(x2, Wq, K_ext, V_ext, Wo)
    return out2.reshape(1, SQ, DM)


# device time: 231137 ns/iter; 1.5435x vs baseline; 1.5435x over previous
import jax
import jax.numpy as jnp
from jax import lax
from jax.experimental import pallas as pl
from jax.experimental.pallas import tpu as pltpu

N_DEV = 4
SQ = 2048
SKV = 2048
DM = 1024
HC = 8
DH = 128
BQ = 256
GW = 128
WIN = 512
NQB = SQ // BQ
SCALE = 0.08838834764831843
F32 = jnp.float32
BF16 = jnp.bfloat16


def kernel(x, Wq, K_ext, V_ext, Wo):
    x2 = (x.reshape(SQ, DM) * SCALE).astype(BF16)
    wq16 = Wq.astype(BF16)
    wo16 = Wo.astype(BF16)

    def body(x_ref, wq_hbm, k_hbm, v_hbm, wo_hbm, out_ref,
             comm, kbuf, k16, vbuf, kwin, vwin, ctxb,
             bias0b, biasmb, biaslb,
             send_sems, recv_sems, kv_sems, w_sems):
        my = lax.axis_index("i")
        left = lax.rem(my + N_DEV - 1, N_DEV)
        right = lax.rem(my + 1, N_DEV)

        cw = pltpu.make_async_copy(wq_hbm, comm.at[0, :DM, :], w_sems.at[0])
        co = pltpu.make_async_copy(wo_hbm, comm.at[0, DM:, :], w_sems.at[1])
        cw.start()
        co.start()

        def kv_copies(h):
            j = lax.rem(my - h + N_DEV, N_DEV)
            cps = []
            for hd in range(HC):
                cps.append(pltpu.make_async_copy(
                    k_hbm.at[my, :, j * HC + hd, :], kbuf.at[hd],
                    kv_sems.at[0]))
                cps.append(pltpu.make_async_copy(
                    v_hbm.at[my, :, j * HC + hd, :], vbuf.at[hd],
                    kv_sems.at[1]))
            return cps

        kv_pend = kv_copies(0)
        for cp in kv_pend:
            cp.start()

        barrier = pltpu.get_barrier_semaphore()
        for nbr in (left, right):
            pl.semaphore_signal(barrier, inc=1, device_id=(nbr,),
                                device_id_type=pl.DeviceIdType.MESH)
        pl.semaphore_wait(barrier, 2)

        cw.wait()
        co.wait()
        for cp in kv_pend:
            cp.wait()
        k16[...] = kbuf[...].astype(BF16)

        r0 = lax.broadcasted_iota(jnp.int32, (BQ, SKV), 0)
        c0 = lax.broadcasted_iota(jnp.int32, (BQ, SKV), 1)
        m0 = (jnp.abs(r0 - c0) <= 128) | (c0 < 32) | (r0 < 32)
        bias0b[...] = jnp.where(m0, 0.0, -1e9).astype(F32)
        rw = lax.broadcasted_iota(jnp.int32, (BQ, GW + WIN), 0)
        cw_ = lax.broadcasted_iota(jnp.int32, (BQ, GW + WIN), 1)
        dmid = (cw_ - GW) - 128 - rw
        mmid = (cw_ < 32) | ((cw_ >= GW) & (jnp.abs(dmid) <= 128))
        biasmb[...] = jnp.where(mmid, 0.0, -1e9).astype(F32)
        dlast = (cw_ - GW) - 256 - rw
        mlast = (cw_ < 32) | ((cw_ >= GW) & (jnp.abs(dlast) <= 128))
        biaslb[...] = jnp.where(mlast, 0.0, -1e9).astype(F32)

        def attend(qh, k_sl, v_sl, bias):
            s = lax.dot_general(qh, k_sl, (((1,), (1,)), ((), ())),
                                preferred_element_type=F32)
            s = s + bias
            mx = jnp.max(s, axis=1, keepdims=True)
            w = jnp.exp(s - mx)
            denom = jnp.sum(w, axis=1, keepdims=True)
            ctx = lax.dot_general(w, v_sl, (((1,), (0,)), ((), ())),
                                  preferred_element_type=F32)
            return ctx / denom

        for h in range(N_DEV):
            slot = h % 2
            if h < N_DEV - 1:
                rdma = pltpu.make_async_remote_copy(
                    src_ref=comm.at[slot],
                    dst_ref=comm.at[1 - slot],
                    send_sem=send_sems.at[slot],
                    recv_sem=recv_sems.at[1 - slot],
                    device_id=(right,),
                    device_id_type=pl.DeviceIdType.MESH,
                )
                rdma.start()

            def accum(qs):
                contrib = lax.dot_general(
                    ctxb[...], comm[slot, DM:, :], (((1,), (0,)), ((), ())),
                    preferred_element_type=F32)
                if h == 0:
                    out_ref[pl.ds(qs, BQ), :] = contrib
                else:
                    out_ref[pl.ds(qs, BQ), :] = (
                        out_ref[pl.ds(qs, BQ), :] + contrib)

            def windowed_block(qs, start, bias):
                q_blk = lax.dot_general(
                    x_ref[pl.ds(qs, BQ), :], comm[slot, :DM, :],
                    (((1,), (0,)), ((), ())),
                    preferred_element_type=F32).astype(BF16)
                for hd in range(HC):
                    kwin[:GW, :] = k16[hd, :GW, :]
                    kwin[GW:, :] = k16[hd, pl.ds(start, WIN), :]
                    vwin[:GW, :] = vbuf[hd, :GW, :]
                    vwin[GW:, :] = vbuf[hd, pl.ds(start, WIN), :]
                    ctxb[:, hd * DH:(hd + 1) * DH] = attend(
                        q_blk[:, hd * DH:(hd + 1) * DH],
                        kwin[...], vwin[...], bias).astype(BF16)
                accum(qs)

            q0 = lax.dot_general(
                x_ref[:BQ, :], comm[slot, :DM, :], (((1,), (0,)), ((), ())),
                preferred_element_type=F32).astype(BF16)
            for hd in range(HC):
                ctxb[:, hd * DH:(hd + 1) * DH] = attend(
                    q0[:, hd * DH:(hd + 1) * DH], k16[hd], vbuf[hd],
                    bias0b[...]).astype(BF16)
            accum(0)

            def qb_body(qb, carry):
                qs = qb * BQ
                windowed_block(qs, qs - 128, biasmb[...])
                return carry

            lax.fori_loop(1, NQB - 1, qb_body, 0)

            windowed_block((NQB - 1) * BQ, SKV - WIN, biaslb[...])

            if h < N_DEV - 1:
                rdma.wait()
                kv_pend = kv_copies(h + 1)
                for cp in kv_pend:
                    cp.start()
                for cp in kv_pend:
                    cp.wait()
                k16[...] = kbuf[...].astype(BF16)

    out2 = pl.pallas_call(
        body,
        out_shape=jax.ShapeDtypeStruct((SQ, DM), F32),
        in_specs=[
            pl.BlockSpec(memory_space=pltpu.VMEM),
            pl.BlockSpec(memory_space=pl.ANY),
            pl.BlockSpec(memory_space=pl.ANY),
            pl.BlockSpec(memory_space=pl.ANY),
            pl.BlockSpec(memory_space=pl.ANY),
        ],
        out_specs=pl.BlockSpec(memory_space=pltpu.VMEM),
        scratch_shapes=[
            pltpu.VMEM((2, 2 * DM, DM), BF16),
            pltpu.VMEM((HC, SKV, DH), F32),
            pltpu.VMEM((HC, SKV, DH), BF16),
            pltpu.VMEM((HC, SKV, DH), F32),
            pltpu.VMEM((GW + WIN, DH), BF16),
            pltpu.VMEM((GW + WIN, DH), F32),
            pltpu.VMEM((BQ, DM), BF16),
            pltpu.VMEM((BQ, SKV), F32),
            pltpu.VMEM((BQ, GW + WIN), F32),
            pltpu.VMEM((BQ, GW + WIN), F32),
            pltpu.SemaphoreType.DMA((2,)),
            pltpu.SemaphoreType.DMA((2,)),
            pltpu.SemaphoreType.DMA((2,)),
            pltpu.SemaphoreType.DMA((2,)),
        ],
        compiler_params=pltpu.CompilerParams(
            collective_id=0,
            vmem_limit_bytes=100 * 1024 * 1024,
        ),
    )(x2, wq16, K_ext, V_ext, wo16)
    return out2.reshape(1, SQ, DM)


# device time: 206092 ns/iter; 1.7311x vs baseline; 1.1215x over previous
import jax
import jax.numpy as jnp
from jax import lax
from jax.experimental import pallas as pl
from jax.experimental.pallas import tpu as pltpu

N_DEV = 4
SQ = 2048
SKV = 2048
DM = 1024
HC = 8
DH = 128
BQ = 256
GW = 128
WIN = 512
NQB = SQ // BQ
SCALE = 0.08838834764831843
F32 = jnp.float32
BF16 = jnp.bfloat16


def kernel(x, Wq, K_ext, V_ext, Wo):
    x2 = (x.reshape(SQ, DM) * SCALE).astype(BF16)
    wq16 = Wq.astype(BF16)
    wo16 = Wo.astype(BF16)

    def body(x_ref, wq_hbm, k_hbm, v_hbm, wo_hbm, out_ref,
             comm, kbuf, vbuf, k16, v16, kwin, vwin, ctxb,
             bias0b, biasmb, biaslb,
             send_sems, recv_sems, kv_sems, w_sems):
        my = lax.axis_index("i")
        left = lax.rem(my + N_DEV - 1, N_DEV)
        right = lax.rem(my + 1, N_DEV)

        cwq = pltpu.make_async_copy(wq_hbm, comm.at[0, :DM, :], w_sems.at[0])
        cwo = pltpu.make_async_copy(wo_hbm, comm.at[0, DM:, :], w_sems.at[1])
        cwq.start()
        cwo.start()

        def kv_copies(h):
            j = lax.rem(my - h + N_DEV, N_DEV)
            cps = []
            for hd in range(HC):
                cps.append(pltpu.make_async_copy(
                    k_hbm.at[my, :, j * HC + hd, :], kbuf.at[hd],
                    kv_sems.at[0]))
                cps.append(pltpu.make_async_copy(
                    v_hbm.at[my, :, j * HC + hd, :], vbuf.at[hd],
                    kv_sems.at[1]))
            return cps

        def kv_wait_convert(cps):
            for cp in cps:
                cp.wait()
            k16[...] = kbuf[...].astype(BF16)
            v16[...] = vbuf[...].astype(BF16)

        kv_pend = kv_copies(0)
        for cp in kv_pend:
            cp.start()

        barrier = pltpu.get_barrier_semaphore()
        for nbr in (left, right):
            pl.semaphore_signal(barrier, inc=1, device_id=(nbr,),
                                device_id_type=pl.DeviceIdType.MESH)
        pl.semaphore_wait(barrier, 2)

        cwq.wait()
        cwo.wait()
        kv_wait_convert(kv_pend)

        r0 = lax.broadcasted_iota(jnp.int32, (BQ, SKV), 0)
        c0 = lax.broadcasted_iota(jnp.int32, (BQ, SKV), 1)
        m0 = (jnp.abs(r0 - c0) <= 128) | (c0 < 32) | (r0 < 32)
        bias0b[...] = jnp.where(m0, 0.0, -1e9).astype(F32)
        rw = lax.broadcasted_iota(jnp.int32, (BQ, GW + WIN), 0)
        cw = lax.broadcasted_iota(jnp.int32, (BQ, GW + WIN), 1)
        dmid = (cw - GW) - 128 - rw
        mmid = (cw < 32) | ((cw >= GW) & (jnp.abs(dmid) <= 128))
        biasmb[...] = jnp.where(mmid, 0.0, -1e9).astype(F32)
        dlast = (cw - GW) - 256 - rw
        mlast = (cw < 32) | ((cw >= GW) & (jnp.abs(dlast) <= 128))
        biaslb[...] = jnp.where(mlast, 0.0, -1e9).astype(F32)

        def attend(qh, k_sl, v_sl, bias):
            s = lax.dot_general(qh, k_sl, (((1,), (1,)), ((), ())),
                                preferred_element_type=F32)
            w = jnp.exp(s + bias)
            denom = jnp.sum(w, axis=1, keepdims=True)
            ctx = lax.dot_general(w.astype(BF16), v_sl,
                                  (((1,), (0,)), ((), ())),
                                  preferred_element_type=F32)
            return ctx / denom

        for h in range(N_DEV):
            slot = h % 2
            if h < N_DEV - 1:
                rdma = pltpu.make_async_remote_copy(
                    src_ref=comm.at[slot],
                    dst_ref=comm.at[1 - slot],
                    send_sem=send_sems.at[slot],
                    recv_sem=recv_sems.at[1 - slot],
                    device_id=(right,),
                    device_id_type=pl.DeviceIdType.MESH,
                )
                rdma.start()
                kv_pend = kv_copies(h + 1)
                for cp in kv_pend:
                    cp.start()

            def accum(qs):
                contrib = lax.dot_general(
                    ctxb[...], comm[slot, DM:, :], (((1,), (0,)), ((), ())),
                    preferred_element_type=F32)
                if h == 0:
                    out_ref[pl.ds(qs, BQ), :] = contrib
                else:
                    out_ref[pl.ds(qs, BQ), :] = (
                        out_ref[pl.ds(qs, BQ), :] + contrib)

            def windowed_block(qs, start, bias):
                q_blk = lax.dot_general(
                    x_ref[pl.ds(qs, BQ), :], comm[slot, :DM, :],
                    (((1,), (0,)), ((), ())),
                    preferred_element_type=F32).astype(BF16)
                for hd in range(HC):
                    kwin[:GW, :] = k16[hd, :GW, :]
                    kwin[GW:, :] = k16[hd, pl.ds(start, WIN), :]
                    vwin[:GW, :] = v16[hd, :GW, :]
                    vwin[GW:, :] = v16[hd, pl.ds(start, WIN), :]
                    ctxb[:, hd * DH:(hd + 1) * DH] = attend(
                        q_blk[:, hd * DH:(hd + 1) * DH],
                        kwin[...], vwin[...], bias).astype(BF16)
                accum(qs)

            q0 = lax.dot_general(
                x_ref[:BQ, :], comm[slot, :DM, :], (((1,), (0,)), ((), ())),
                preferred_element_type=F32).astype(BF16)
            for hd in range(HC):
                ctxb[:, hd * DH:(hd + 1) * DH] = attend(
                    q0[:, hd * DH:(hd + 1) * DH], k16[hd], v16[hd],
                    bias0b[...]).astype(BF16)
            accum(0)

            def qb_body(qb, carry):
                qs = qb * BQ
                windowed_block(qs, qs - 128, biasmb[...])
                return carry

            lax.fori_loop(1, NQB - 1, qb_body, 0)

            windowed_block((NQB - 1) * BQ, SKV - WIN, biaslb[...])

            if h < N_DEV - 1:
                rdma.wait()
                kv_wait_convert(kv_pend)

    out2 = pl.pallas_call(
        body,
        out_shape=jax.ShapeDtypeStruct((SQ, DM), F32),
        in_specs=[
            pl.BlockSpec(memory_space=pltpu.VMEM),
            pl.BlockSpec(memory_space=pl.ANY),
            pl.BlockSpec(memory_space=pl.ANY),
            pl.BlockSpec(memory_space=pl.ANY),
            pl.BlockSpec(memory_space=pl.ANY),
        ],
        out_specs=pl.BlockSpec(memory_space=pltpu.VMEM),
        scratch_shapes=[
            pltpu.VMEM((2, 2 * DM, DM), BF16),
            pltpu.VMEM((HC, SKV, DH), F32),
            pltpu.VMEM((HC, SKV, DH), F32),
            pltpu.VMEM((HC, SKV, DH), BF16),
            pltpu.VMEM((HC, SKV, DH), BF16),
            pltpu.VMEM((GW + WIN, DH), BF16),
            pltpu.VMEM((GW + WIN, DH), BF16),
            pltpu.VMEM((BQ, DM), BF16),
            pltpu.VMEM((BQ, SKV), F32),
            pltpu.VMEM((BQ, GW + WIN), F32),
            pltpu.VMEM((BQ, GW + WIN), F32),
            pltpu.SemaphoreType.DMA((2,)),
            pltpu.SemaphoreType.DMA((2,)),
            pltpu.SemaphoreType.DMA((2,)),
            pltpu.SemaphoreType.DMA((2,)),
        ],
        compiler_params=pltpu.CompilerParams(
            collective_id=0,
            vmem_limit_bytes=100 * 1024 * 1024,
        ),
    )(x2, wq16, K_ext, V_ext, wo16)
    return out2.reshape(1, SQ, DM)
